# baseline (speedup 1.0000x reference)
import functools

import jax
import jax.numpy as jnp
from jax import lax
from jax.experimental import pallas as pl
from jax.experimental.pallas import tpu as pltpu
from jax.experimental.pallas import tpu_sc as plsc

NC = 2
NS = 16
NW = NC * NS
LANES = 16

ARG_SEQS = 8


def _ceil16_offsets(ctx):
  offs = []
  k = 0
  while k + LANES <= ctx:
    offs.append(k)
    k += LANES
  if k < ctx:
    offs.append(ctx - LANES)
  return offs


def _b_body(v, dim, b, ctx, tt_hbm, text_hbm, tail_hbm, tp_out, len_out,
            blk_in_a, blk_in_b, blk_out_a, blk_out_b, idx_a, idx_b, len_v,
            tail_v, sem_bi_a, sem_bi_b, sem_bo_a, sem_bo_b, sem_idx_a,
            sem_idx_b):
  wid = lax.axis_index("s") * NC + lax.axis_index("c")
  iota = lax.broadcasted_iota(jnp.int32, (LANES,), 0)
  tail_offs = _ceil16_offsets(ctx)

  seqs_w = b // NW
  n_arg = seqs_w // ARG_SEQS
  arg_rows = ARG_SEQS * ctx
  w_seq0 = wid * seqs_w

  def start_idx(i, ref, sem):
    base = (w_seq0 + i * ARG_SEQS) * ctx
    pltpu.async_copy(text_hbm.at[pl.ds(base, arg_rows)], ref, sem)

  def drain_idx(ref, sem):
    pltpu.make_async_copy(text_hbm.at[pl.ds(0, arg_rows)], ref, sem).wait()

  def reduce16(vec, op):
    m = vec[0]
    for k in range(1, LANES):
      m = op(m, vec[k])
    return m

  def seq_argmax(idx_ref, base):
    vmax = idx_ref[pl.ds(base + tail_offs[0], LANES)]
    for off in tail_offs[1:]:
      vmax = jnp.maximum(vmax, idx_ref[pl.ds(base + off, LANES)])
    m = reduce16(vmax, jnp.maximum)
    vpos = jnp.full((LANES,), jnp.int32(0x7FFFFFFF), dtype=jnp.int32)
    for off in tail_offs:
      vv = idx_ref[pl.ds(base + off, LANES)]
      vpos = jnp.minimum(vpos, jnp.where(vv == m, iota + off, 0x7FFFFFFF))
    return reduce16(vpos, jnp.minimum)

  def arg_chunk(i, idx_ref, sem, acc):
    drain_idx(idx_ref, sem)
    lane0 = (i % 2) * ARG_SEQS
    for s in range(ARG_SEQS):
      r = seq_argmax(idx_ref, s * ctx)
      acc = jnp.where(iota == lane0 + s, r, acc)

    @pl.when(i % 2 == 1)
    def _():
      len_v[pl.ds((i // 2) * LANES, LANES)] = acc

    return acc

  start_idx(0, idx_a, sem_idx_a)

  def arg_outer(i, acc):
    start_idx(2 * i + 1, idx_b, sem_idx_b)
    acc = arg_chunk(2 * i, idx_a, sem_idx_a, acc)

    @pl.when(2 * i + 2 < n_arg)
    def _():
      start_idx(2 * i + 2, idx_a, sem_idx_a)

    acc = arg_chunk(2 * i + 1, idx_b, sem_idx_b, acc)
    return acc

  lax.fori_loop(0, n_arg // 2, arg_outer, jnp.zeros((LANES,), jnp.int32))
  pltpu.sync_copy(len_v, len_out.at[pl.ds(wid * seqs_w, seqs_w)])

  n_full = v // 128
  n_w = (n_full - wid + NW - 1) // NW

  dim_idx = [iota + ((16 * g) % 64) for g in range(dim * 2 // LANES)]

  def blk_v0(k):
    return pl.multiple_of((wid + k * NW) * 128, 128)

  def start_blk_in(k, ref, sem):
    pltpu.async_copy(tt_hbm.at[:, pl.ds(blk_v0(k), 128)], ref, sem)

  def drain_blk_in(ref, sem):
    pltpu.make_async_copy(tt_hbm.at[:, pl.ds(0, 128)], ref, sem).wait()

  def transpose_blk(src, dst, nrows):
    for r in range(nrows):
      for g in range(dim * 2 // LANES):
        row = 2 * r + (1 if g >= dim // LANES else 0)
        vals = plsc.load_gather(src, [dim_idx[g], jnp.full((LANES,), row,
                                                           jnp.int32)])
        dst[r, pl.ds(16 * g, LANES)] = vals

  def start_blk_out(k, ref, sem):
    off = pl.multiple_of(blk_v0(k) // 2, 64)
    pltpu.async_copy(ref, tp_out.at[pl.ds(off, 64)], sem)

  def drain_blk_out(ref, sem):
    pltpu.make_async_copy(ref, tp_out.at[pl.ds(0, 64)], sem).wait()

  start_blk_in(0, blk_in_a, sem_bi_a)

  def t_chunk(k, blk_in, blk_out, s_in, s_out, other_in, s_other_in):
    @pl.when(k + 1 < n_w)
    def _():
      start_blk_in(k + 1, other_in, s_other_in)

    drain_blk_in(blk_in, s_in)

    @pl.when(k >= 2)
    def _():
      drain_blk_out(blk_out, s_out)

    transpose_blk(blk_in, blk_out, 64)
    start_blk_out(k, blk_out, s_out)

  def t_outer(i, carry):
    @pl.when(2 * i < n_w)
    def _():
      t_chunk(2 * i, blk_in_a, blk_out_a, sem_bi_a, sem_bo_a,
              blk_in_b, sem_bi_b)

    @pl.when(2 * i + 1 < n_w)
    def _():
      t_chunk(2 * i + 1, blk_in_b, blk_out_b, sem_bi_b, sem_bo_b,
              blk_in_a, sem_bi_a)

    return carry

  lax.fori_loop(0, (n_w + 1) // 2, t_outer, 0)

  @pl.when(n_w >= 1)
  def _():
    drain_blk_out(blk_out_a, sem_bo_a)

  @pl.when(n_w >= 2)
  def _():
    drain_blk_out(blk_out_b, sem_bo_b)

  tail_cols = v - n_full * 128
  if tail_cols:
    @pl.when(wid == n_full % NW)
    def _():
      pltpu.sync_copy(tail_hbm, tail_v)
      pltpu.sync_copy(tail_v, tp_out.at[pl.ds(n_full * 64,
                                              tail_cols // 2)])


def _c_body(v, dim, b, ctx, tp_hbm, text_t_hbm, pos_hbm, ot_out,
            idxall, pair_a, pair_b, par_a, par_b, g_a, g_b, o_a, o_b,
            pos_v, sem_g_a, sem_g_b, sem_o_a, sem_o_b, sem_p):
  wid = lax.axis_index("s") * NC + lax.axis_index("c")
  iota = lax.broadcasted_iota(jnp.int32, (LANES,), 0)

  b0 = pl.multiple_of(wid * 128, 128)
  pltpu.async_copy(pos_hbm, pos_v, sem_p)
  pltpu.async_copy(text_t_hbm.at[:, pl.ds(b0, 128)], idxall,
                   sem_g_a).wait()
  pltpu.make_async_copy(pos_hbm, pos_v, sem_p).wait()

  def make_pairs(t, pair_ref, par_ref):
    for g in range(128 // LANES):
      ii = idxall[t, pl.ds(16 * g, LANES)]
      pair_ref[pl.ds(16 * g, LANES)] = ii >> 1
      par_ref[pl.ds(16 * g, LANES)] = ii & 1

  def start_gather(pair_ref, g_ref, sem):
    pltpu.async_copy(tp_hbm.at[pair_ref], g_ref, sem)

  def drain_gather(g_ref, sem):
    pltpu.make_async_copy(tp_hbm.at[pl.ds(0, 128)], g_ref, sem).wait()

  def assemble(t, g_ref, o_ref, par_ref):
    prow = [pos_v[t, pl.ds(16 * q, LANES)] for q in range(dim // LANES)]
    coffs = [par_ref[pl.ds(16 * g, LANES)] * dim for g in range(8)]
    for d in range(dim):
      p = prow[d // LANES][d % LANES]
      for g in range(128 // LANES):
        jvec = iota + 16 * g
        vals = plsc.load_gather(g_ref, [jvec, coffs[g] + d])
        o_ref[d, pl.ds(16 * g, LANES)] = vals + p

  def start_out(t, o_ref, sem):
    pltpu.async_copy(o_ref, ot_out.at[t, :, pl.ds(b0, 128)], sem)

  def drain_out(o_ref, sem):
    pltpu.make_async_copy(o_ref, ot_out.at[0, :, pl.ds(0, 128)],
                          sem).wait()

  pairs_b = (pair_a, pair_b)
  parbufs = (par_a, par_b)
  gbufs = (g_a, g_b)
  obufs = (o_a, o_b)
  sems_g = (sem_g_a, sem_g_b)
  sems_o = (sem_o_a, sem_o_b)

  def stage1(t, p):
    make_pairs(t, pairs_b[p], parbufs[p])
    start_gather(pairs_b[p], gbufs[p], sems_g[p])

  def stage2(t, p):
    drain_gather(gbufs[p], sems_g[p])

    @pl.when(t >= 2)
    def _():
      drain_out(obufs[p], sems_o[p])

    assemble(t, gbufs[p], obufs[p], parbufs[p])
    start_out(t, obufs[p], sems_o[p])

  def outer2(i, carry):
    stage1(2 * i, 0)

    @pl.when(2 * i - 1 >= 0)
    def _():
      stage2(2 * i - 1, 1)

    stage1(2 * i + 1, 1)
    stage2(2 * i, 0)
    return carry

  lax.fori_loop(0, ctx // 2, outer2, 0)
  stage2(ctx - 1, 1)
  drain_out(obufs[0], sems_o[0])
  drain_out(obufs[1], sems_o[1])


def _mask_body(ctx, o_ref):
  r = lax.broadcasted_iota(jnp.int32, (ctx, ctx), 0)
  c = lax.broadcasted_iota(jnp.int32, (ctx, ctx), 1)
  o_ref[...] = jnp.where(c > r, -jnp.inf, 0.0).astype(jnp.float32)


def kernel(text, token_embedding, pos_embed):
  b, ctx = text.shape
  v, dim = token_embedding.shape
  assert b % NW == 0 and (b // NW) % (2 * ARG_SEQS) == 0
  assert dim == 64 and b // 128 == NW and v % 2 == 0 and v >= 128

  tt = token_embedding.T
  text_t = text.T
  text_flat = text.reshape(b * ctx).astype(jnp.int32)
  pos2d = pos_embed.reshape(ctx, dim)

  mesh = plsc.VectorSubcoreMesh(core_axis_name="c", subcore_axis_name="s")
  tiled = pltpu.CompilerParams(use_tc_tiling_on_sc=True,
                               needs_layout_passes=False)

  kb = pl.kernel(
      functools.partial(_b_body, v, dim, b, ctx),
      out_type=(
          jax.ShapeDtypeStruct((v // 2, 2 * dim), jnp.float32),
          jax.ShapeDtypeStruct((b,), jnp.int32),
      ),
      mesh=mesh,
      compiler_params=tiled,
      scratch_types=[
          pltpu.VMEM((dim, 128), jnp.float32),
          pltpu.VMEM((dim, 128), jnp.float32),
          pltpu.VMEM((64, 2 * dim), jnp.float32),
          pltpu.VMEM((64, 2 * dim), jnp.float32),
          pltpu.VMEM((ARG_SEQS * ctx,), jnp.int32),
          pltpu.VMEM((ARG_SEQS * ctx,), jnp.int32),
          pltpu.VMEM((b // NW,), jnp.int32),
          pltpu.VMEM((max(v % 128, 2) // 2, 2 * dim), jnp.float32),
          pltpu.SemaphoreType.DMA,
          pltpu.SemaphoreType.DMA,
          pltpu.SemaphoreType.DMA,
          pltpu.SemaphoreType.DMA,
          pltpu.SemaphoreType.DMA,
          pltpu.SemaphoreType.DMA,
      ],
  )
  tail = v % 128
  if tail:
    tail128 = lax.slice(token_embedding, (v - tail, 0),
                        (v, dim)).reshape(tail // 2, 2 * dim)
  else:
    tail128 = jnp.zeros((1, 2 * dim), jnp.float32)
  table_pairs, lengths = kb(tt, text_flat, tail128)

  kc = pl.kernel(
      functools.partial(_c_body, v, dim, b, ctx),
      out_type=jax.ShapeDtypeStruct((ctx, dim, b), jnp.float32),
      mesh=mesh,
      compiler_params=tiled,
      scratch_types=[
          pltpu.VMEM((ctx, 128), jnp.int32),
          pltpu.VMEM((128,), jnp.int32),
          pltpu.VMEM((128,), jnp.int32),
          pltpu.VMEM((128,), jnp.int32),
          pltpu.VMEM((128,), jnp.int32),
          pltpu.VMEM((128, 2 * dim), jnp.float32),
          pltpu.VMEM((128, 2 * dim), jnp.float32),
          pltpu.VMEM((dim, 128), jnp.float32),
          pltpu.VMEM((dim, 128), jnp.float32),
          pltpu.VMEM((ctx, dim), jnp.float32),
          pltpu.SemaphoreType.DMA,
          pltpu.SemaphoreType.DMA,
          pltpu.SemaphoreType.DMA,
          pltpu.SemaphoreType.DMA,
          pltpu.SemaphoreType.DMA,
      ],
  )
  out_t = kc(table_pairs, text_t, pos2d)
  token_text = out_t.transpose(2, 0, 1)

  mask = pl.pallas_call(
      functools.partial(_mask_body, ctx),
      out_shape=jax.ShapeDtypeStruct((ctx, ctx), jnp.float32),
  )()

  return token_text, lengths, mask


# speedup vs baseline: 2.2202x; 2.2202x over previous
import functools

import jax
import jax.numpy as jnp
from jax import lax
from jax.experimental import pallas as pl
from jax.experimental.pallas import tpu as pltpu
from jax.experimental.pallas import tpu_sc as plsc

NC = 2
NS = 16
NW = NC * NS
LANES = 16

CHUNK_SEQS = 4


def _ceil16_offsets(ctx):
  offs = []
  k = 0
  while k + LANES <= ctx:
    offs.append(k)
    k += LANES
  if k < ctx:
    offs.append(ctx - LANES)
  return offs


def _sc_body(ctx, dim, n_chunks, chunk_rows, text_hbm, table_hbm, pos_hbm,
             emb_out, len_out, idx_a, idx_b, rows_a, rows_b, pos_v, len_v,
             sem_idx_a, sem_idx_b, sem_g_a, sem_g_b, sem_o_a,
             sem_o_b, sem_p):
  wid = lax.axis_index("s") * NC + lax.axis_index("c")
  w_row0 = wid * (n_chunks * chunk_rows)

  pltpu.async_copy(pos_hbm, pos_v, sem_p).wait()

  iota = lax.broadcasted_iota(jnp.int32, (LANES,), 0)
  tail_offs = _ceil16_offsets(ctx)

  lo = (ctx // 2 + 7) // 8 * 8
  hi = ctx - lo
  assert lo % 8 == 0 and lo <= 128 and hi <= 128

  def start_idx(c, idx_ref, sem):
    base = w_row0 + c * chunk_rows
    pltpu.async_copy(text_hbm.at[pl.ds(base, chunk_rows)], idx_ref, sem)

  def drain_idx(idx_ref, sem):
    pltpu.make_async_copy(text_hbm.at[pl.ds(0, chunk_rows)], idx_ref,
                          sem).wait()

  def start_gather(idx_ref, rows_ref, sem):
    for s in range(CHUNK_SEQS):
      b = s * ctx
      pltpu.async_copy(table_hbm.at[idx_ref.at[pl.ds(b, lo)]],
                       rows_ref.at[pl.ds(b, lo)], sem)
      pltpu.async_copy(table_hbm.at[idx_ref.at[pl.ds(b + lo, hi)]],
                       rows_ref.at[pl.ds(b + lo, hi)], sem)

  def drain_gather(rows_ref, sem):
    pltpu.make_async_copy(table_hbm.at[pl.ds(0, chunk_rows)], rows_ref,
                          sem).wait()

  def start_out(c, rows_ref, sem):
    base = w_row0 + c * chunk_rows
    pltpu.async_copy(rows_ref, emb_out.at[pl.ds(base, chunk_rows)], sem)

  def drain_out(rows_ref, sem):
    pltpu.make_async_copy(rows_ref, emb_out.at[pl.ds(0, chunk_rows)],
                          sem).wait()

  def reduce16(vec, op):
    m = vec[0]
    for k in range(1, LANES):
      m = op(m, vec[k])
    return m

  def seq_argmax(idx_ref, base):
    vmax = idx_ref[pl.ds(base + tail_offs[0], LANES)]
    for off in tail_offs[1:]:
      vmax = jnp.maximum(vmax, idx_ref[pl.ds(base + off, LANES)])
    m = reduce16(vmax, jnp.maximum)
    vpos = jnp.full((LANES,), jnp.int32(0x7FFFFFFF), dtype=jnp.int32)
    for off in tail_offs:
      v = idx_ref[pl.ds(base + off, LANES)]
      vpos = jnp.minimum(vpos, jnp.where(v == m, iota + off, 0x7FFFFFFF))
    return reduce16(vpos, jnp.minimum)

  def add_pos(rows_ref):
    def body(t, carry):
      for s in range(CHUNK_SEQS):
        for q in range(dim // LANES):
          sl = pl.ds(q * LANES, LANES)
          p = pos_v[t, sl]
          rows_ref[s * ctx + t, sl] = rows_ref[s * ctx + t, sl] + p
      return carry
    lax.fori_loop(0, ctx, body, 0, unroll=2)

  def do_chunk(c, idx_cur, rows_cur, s_idx, s_g, s_o, idx_nxt, s_idx_nxt,
               rows_prev_free, acc):
    drain_idx(idx_cur, s_idx)

    @pl.when(rows_prev_free)
    def _():
      drain_out(rows_cur, s_o)

    start_gather(idx_cur, rows_cur, s_g)

    @pl.when(c + 1 < n_chunks)
    def _():
      start_idx(c + 1, idx_nxt, s_idx_nxt)

    lane0 = (c % 4) * CHUNK_SEQS
    for s in range(CHUNK_SEQS):
      r = seq_argmax(idx_cur, s * ctx)
      acc = jnp.where(iota == lane0 + s, r, acc)

    @pl.when(c % 4 == 3)
    def _():
      len_v[pl.ds((c // 4) * LANES, LANES)] = acc

    drain_gather(rows_cur, s_g)
    add_pos(rows_cur)
    start_out(c, rows_cur, s_o)
    return acc

  start_idx(0, idx_a, sem_idx_a)

  def outer(i, acc):
    acc = do_chunk(2 * i, idx_a, rows_a, sem_idx_a, sem_g_a, sem_o_a,
                   idx_b, sem_idx_b, i >= 1, acc)
    acc = do_chunk(2 * i + 1, idx_b, rows_b, sem_idx_b, sem_g_b, sem_o_b,
                   idx_a, sem_idx_a, i >= 1, acc)
    return acc

  lax.fori_loop(0, n_chunks // 2, outer, jnp.zeros((LANES,), jnp.int32))

  drain_out(rows_a, sem_o_a)
  drain_out(rows_b, sem_o_b)

  n_seq_w = n_chunks * CHUNK_SEQS
  pltpu.sync_copy(len_v, len_out.at[pl.ds(wid * n_seq_w, n_seq_w)])


def _mask_body(ctx, o_ref):
  r = lax.broadcasted_iota(jnp.int32, (ctx, ctx), 0)
  c = lax.broadcasted_iota(jnp.int32, (ctx, ctx), 1)
  o_ref[...] = jnp.where(c > r, -jnp.inf, 0.0).astype(jnp.float32)


def kernel(text, token_embedding, pos_embed):
  b, ctx = text.shape
  _, dim = token_embedding.shape

  rows_total = b * ctx
  assert rows_total % NW == 0
  rows_per_w = rows_total // NW
  chunk_rows = CHUNK_SEQS * ctx
  assert rows_per_w % chunk_rows == 0
  n_chunks = rows_per_w // chunk_rows
  assert n_chunks % 2 == 0
  assert dim % LANES == 0

  text_flat = text.reshape(rows_total).astype(jnp.int32)
  pos2d = pos_embed.reshape(ctx, dim)

  mesh = plsc.VectorSubcoreMesh(core_axis_name="c", subcore_axis_name="s")
  sc = pl.kernel(
      functools.partial(_sc_body, ctx, dim, n_chunks, chunk_rows),
      out_type=(
          jax.ShapeDtypeStruct((rows_total, dim), jnp.float32),
          jax.ShapeDtypeStruct((b,), jnp.int32),
      ),
      mesh=mesh,
      compiler_params=pltpu.CompilerParams(use_tc_tiling_on_sc=False),
      scratch_types=[
          pltpu.VMEM((chunk_rows,), jnp.int32),
          pltpu.VMEM((chunk_rows,), jnp.int32),
          pltpu.VMEM((chunk_rows, dim), jnp.float32),
          pltpu.VMEM((chunk_rows, dim), jnp.float32),
          pltpu.VMEM((ctx, dim), jnp.float32),
          pltpu.VMEM((rows_per_w // ctx,), jnp.int32),
          pltpu.SemaphoreType.DMA,
          pltpu.SemaphoreType.DMA,
          pltpu.SemaphoreType.DMA,
          pltpu.SemaphoreType.DMA,
          pltpu.SemaphoreType.DMA,
          pltpu.SemaphoreType.DMA,
          pltpu.SemaphoreType.DMA,
      ],
  )
  emb_flat, lengths = sc(text_flat, token_embedding, pos2d)
  token_text = emb_flat.reshape(b, ctx, dim)

  mask = pl.pallas_call(
      functools.partial(_mask_body, ctx),
      out_shape=jax.ShapeDtypeStruct((ctx, ctx), jnp.float32),
  )()

  return token_text, lengths, mask
